# baseline (speedup 1.0000x reference)
import functools

import jax
import jax.numpy as jnp
from jax import lax
from jax.experimental import pallas as pl
from jax.experimental.pallas import tpu as pltpu
from jax.experimental.pallas import tpu_sc as plsc

N_NODES = 10000
N_EDGES = 320000
NPAD = 10240
NT = 32
NSUB = 16
SLICE = NPAD // NSUB
L = 16

CHUNK = 128
NCHUNK = N_EDGES // CHUNK
DEG_E = (NCHUNK // NSUB) * CHUNK
SCAT_E = (NCHUNK // NT) * CHUNK
TAIL_OFF = (NCHUNK // NT) * NT * CHUNK
TAIL_E = N_EDGES - TAIL_OFF
DTAIL_OFF = (NCHUNK // NSUB) * NSUB * CHUNK

_RSQRT_MAGIC = 0x5F3759DF


def _zero_tables(refs):
    @plsc.parallel_loop(0, NPAD, step=L, unroll=8)
    def _(i):
        for r in refs:
            r[pl.ds(i, L)] = jnp.zeros((L,), jnp.float32)


def _sc_mega(edges, g0, g1):
    mesh = plsc.VectorSubcoreMesh(core_axis_name="c", subcore_axis_name="s")

    @functools.partial(
        pl.kernel,
        mesh=mesh,
        out_type=[
            jax.ShapeDtypeStruct((2, NPAD), jnp.float32),
            jax.ShapeDtypeStruct((2, NPAD), jnp.float32),
            jax.ShapeDtypeStruct((2, NPAD), jnp.float32),
            jax.ShapeDtypeStruct((NT, NPAD), jnp.float32),
            jax.ShapeDtypeStruct((NT, NPAD), jnp.float32),
        ],
        scratch_types=[
            pltpu.VMEM((DEG_E,), jnp.int32),
            pltpu.VMEM((SCAT_E,), jnp.int32),
            pltpu.VMEM((SCAT_E,), jnp.int32),
            pltpu.VMEM((TAIL_E,), jnp.int32),
            pltpu.VMEM((TAIL_E,), jnp.int32),
            pltpu.VMEM((TAIL_E,), jnp.int32),
            pltpu.VMEM((NPAD,), jnp.float32),
            pltpu.VMEM((NPAD,), jnp.float32),
            pltpu.VMEM((NPAD,), jnp.float32),
            pltpu.VMEM((NPAD,), jnp.float32),
            pltpu.VMEM((NPAD,), jnp.float32),
            pltpu.VMEM((NPAD,), jnp.float32),
            pltpu.VMEM((SLICE,), jnp.float32),
            pltpu.VMEM((SLICE,), jnp.float32),
            pltpu.VMEM((SLICE,), jnp.float32),
            pltpu.VMEM_SHARED((NSUB, NPAD), jnp.float32),
            pltpu.VMEM_SHARED((NPAD,), jnp.float32),
            pltpu.VMEM_SHARED((NPAD,), jnp.float32),
        ],
        compiler_params=pltpu.CompilerParams(needs_layout_passes=False),
    )
    def mega(e_hbm, g0_hbm, g1_hbm,
             dinv_hbm, gs0_hbm, gs1_hbm, o0_hbm, o1_hbm,
             d_v, s_v, d2_v, td_v, ts_v, td2_v, deg, red, t0, t1, a0, a1,
             dv_s, g0s, g1s, sh_deg, sh_g0, sh_g1):
        sid = lax.axis_index("s")
        cid = lax.axis_index("c")

        _zero_tables([deg, a0, a1])
        dbase = pl.multiple_of(sid * DEG_E, CHUNK)
        pltpu.sync_copy(e_hbm.at[1, pl.ds(dbase, DEG_E)], d_v)
        ones = jnp.ones((L,), jnp.float32)

        @plsc.parallel_loop(0, DEG_E, step=L, unroll=8)
        def _(i):
            plsc.addupdate_scatter(deg, [d_v[pl.ds(i, L)]], ones)

        @pl.when(sid == NSUB - 1)
        def _():
            pltpu.sync_copy(e_hbm.at[1, pl.ds(DTAIL_OFF, TAIL_E)], td_v)

            @plsc.parallel_loop(0, TAIL_E, step=L, unroll=8)
            def _(i):
                plsc.addupdate_scatter(deg, [td_v[pl.ds(i, L)]], ones)

        pltpu.sync_copy(deg, sh_deg.at[sid])
        plsc.subcore_barrier()

        off = pl.multiple_of(sid * SLICE, 8)
        for r in range(NSUB):
            pltpu.sync_copy(sh_deg.at[r, pl.ds(off, SLICE)],
                            red.at[pl.ds(r * SLICE, SLICE)])
        pltpu.sync_copy(g0_hbm.at[pl.ds(off, SLICE)], g0s)
        pltpu.sync_copy(g1_hbm.at[pl.ds(off, SLICE)], g1s)

        half = jnp.full((L,), 0.5, jnp.float32)
        three_half = jnp.full((L,), 1.5, jnp.float32)
        magic = jnp.full((L,), _RSQRT_MAGIC, jnp.int32)
        one_i = jnp.full((L,), 1, jnp.int32)

        @plsc.parallel_loop(0, SLICE, step=L, unroll=4)
        def _(j):
            s = red[pl.ds(j, L)]
            for r in range(1, NSUB):
                s = s + red[pl.ds(r * SLICE + j, L)]
            dd = s + 1.0
            y = plsc.bitcast(
                magic - lax.shift_right_logical(plsc.bitcast(dd, jnp.int32),
                                                one_i),
                jnp.float32)
            for _ in range(3):
                y = y * (three_half - half * dd * y * y)
            dv_s[pl.ds(j, L)] = y
            g0s[pl.ds(j, L)] = g0s[pl.ds(j, L)] * y
            g1s[pl.ds(j, L)] = g1s[pl.ds(j, L)] * y

        pltpu.sync_copy(g0s, sh_g0.at[pl.ds(off, SLICE)])
        pltpu.sync_copy(g1s, sh_g1.at[pl.ds(off, SLICE)])
        pltpu.sync_copy(dv_s, dinv_hbm.at[cid, pl.ds(off, SLICE)])
        pltpu.sync_copy(g0s, gs0_hbm.at[cid, pl.ds(off, SLICE)])
        pltpu.sync_copy(g1s, gs1_hbm.at[cid, pl.ds(off, SLICE)])
        plsc.subcore_barrier()
        pltpu.sync_copy(sh_g0, t0)
        pltpu.sync_copy(sh_g1, t1)

        wid = cid * NSUB + sid
        sbase = pl.multiple_of(wid * SCAT_E, CHUNK)
        pltpu.sync_copy(e_hbm.at[0, pl.ds(sbase, SCAT_E)], s_v)
        pltpu.sync_copy(e_hbm.at[1, pl.ds(sbase, SCAT_E)], d2_v)

        @plsc.parallel_loop(0, SCAT_E, step=L, unroll=8)
        def _(i):
            s = s_v[pl.ds(i, L)]
            d = d2_v[pl.ds(i, L)]
            v0 = plsc.load_gather(t0, [s])
            plsc.addupdate_scatter(a0, [d], v0)
            v1 = plsc.load_gather(t1, [s])
            plsc.addupdate_scatter(a1, [d], v1)

        @pl.when(wid == NT - 1)
        def _():
            pltpu.sync_copy(e_hbm.at[0, pl.ds(TAIL_OFF, TAIL_E)], ts_v)
            pltpu.sync_copy(e_hbm.at[1, pl.ds(TAIL_OFF, TAIL_E)], td2_v)

            @plsc.parallel_loop(0, TAIL_E, step=L, unroll=8)
            def _(i):
                s = ts_v[pl.ds(i, L)]
                d = td2_v[pl.ds(i, L)]
                v0 = plsc.load_gather(t0, [s])
                plsc.addupdate_scatter(a0, [d], v0)
                v1 = plsc.load_gather(t1, [s])
                plsc.addupdate_scatter(a1, [d], v1)

        pltpu.sync_copy(a0, o0_hbm.at[wid])
        pltpu.sync_copy(a1, o1_hbm.at[wid])

    return mega(edges, g0, g1)


def _tc_matmul(x, W1, W2):
    def body(x_ref, w1_ref, w2_ref, o0_ref, o1_ref):
        wc = jnp.dot(w1_ref[...], w2_ref[...],
                     preferred_element_type=jnp.float32)
        gt = lax.dot_general(wc, x_ref[...], (((0,), (1,)), ((), ())),
                             preferred_element_type=jnp.float32)
        pad = jnp.zeros((1, NPAD - N_NODES), jnp.float32)
        o0_ref[...] = jnp.concatenate([gt[0:1, :], pad], axis=1)
        o1_ref[...] = jnp.concatenate([gt[1:2, :], pad], axis=1)

    return pl.pallas_call(
        body,
        out_shape=[
            jax.ShapeDtypeStruct((1, NPAD), jnp.float32),
            jax.ShapeDtypeStruct((1, NPAD), jnp.float32),
        ],
    )(x, W1, W2)


def _tc_final(a0p, a1p, gs0, gs1, dinv2, b1r, W2, b2r):
    def body(a0_ref, a1_ref, gs0_ref, gs1_ref, dinv_ref, b1_ref, w2_ref,
             b2_ref, o_ref):
        bc = jnp.dot(b1_ref[...], w2_ref[...],
                     preferred_element_type=jnp.float32) + b2_ref[...]
        d = dinv_ref[0:1, :N_NODES]
        s0 = (jnp.sum(a0_ref[...], axis=0, keepdims=True)
              + gs0_ref[0:1, :])[:, :N_NODES]
        s1 = (jnp.sum(a1_ref[...], axis=0, keepdims=True)
              + gs1_ref[0:1, :])[:, :N_NODES]
        o_ref[...] = jnp.concatenate(
            [d * s0 + bc[:, 0:1], d * s1 + bc[:, 1:2]], axis=0)

    return pl.pallas_call(
        body,
        out_shape=jax.ShapeDtypeStruct((2, N_NODES), jnp.float32),
    )(a0p, a1p, gs0, gs1, dinv2, b1r, W2, b2r)


def kernel(x, edge_index, W1, b1, W2, b2):
    ei = edge_index.astype(jnp.int32)

    g0r, g1r = _tc_matmul(x, W1, W2)
    dinv2, gs0, gs1, a0p, a1p = _sc_mega(
        ei, g0r.reshape(NPAD), g1r.reshape(NPAD))

    out2 = _tc_final(a0p, a1p, gs0, gs1, dinv2,
                     b1.reshape(1, -1), W2, b2.reshape(1, -1))
    return out2.T


# speedup vs baseline: 1.0246x; 1.0246x over previous
import functools

import jax
import jax.numpy as jnp
from jax import lax
from jax.experimental import pallas as pl
from jax.experimental.pallas import tpu as pltpu
from jax.experimental.pallas import tpu_sc as plsc

N_NODES = 10000
N_EDGES = 320000
NPAD = 10240
NT = 32
NSUB = 16
SLICE = NPAD // NSUB
EPC = N_EDGES // NSUB
EPT = N_EDGES // NT
L = 16

_RSQRT_MAGIC = 0x5F3759DF


def _zero_tables(refs):
    @plsc.parallel_loop(0, NPAD, step=L, unroll=8)
    def _(i):
        for r in refs:
            r[pl.ds(i, L)] = jnp.zeros((L,), jnp.float32)


def _sc_mega(src, dst, g0, g1):
    mesh = plsc.VectorSubcoreMesh(core_axis_name="c", subcore_axis_name="s")

    @functools.partial(
        pl.kernel,
        mesh=mesh,
        out_type=[
            jax.ShapeDtypeStruct((2, NPAD), jnp.float32),
            jax.ShapeDtypeStruct((2, NPAD), jnp.float32),
            jax.ShapeDtypeStruct((2, NPAD), jnp.float32),
            jax.ShapeDtypeStruct((NT, NPAD), jnp.float32),
            jax.ShapeDtypeStruct((NT, NPAD), jnp.float32),
        ],
        scratch_types=[
            pltpu.VMEM((EPC,), jnp.int32),
            pltpu.VMEM((EPT,), jnp.int32),
            pltpu.VMEM((NPAD,), jnp.float32),
            pltpu.VMEM((NPAD,), jnp.float32),
            pltpu.VMEM((NPAD,), jnp.float32),
            pltpu.VMEM((NPAD,), jnp.float32),
            pltpu.VMEM((NPAD,), jnp.float32),
            pltpu.VMEM((NPAD,), jnp.float32),
            pltpu.VMEM((SLICE,), jnp.float32),
            pltpu.VMEM((SLICE,), jnp.float32),
            pltpu.VMEM((SLICE,), jnp.float32),
            pltpu.VMEM_SHARED((NSUB, NPAD), jnp.float32),
            pltpu.VMEM_SHARED((NPAD,), jnp.float32),
            pltpu.VMEM_SHARED((NPAD,), jnp.float32),
        ],
        compiler_params=pltpu.CompilerParams(needs_layout_passes=False),
    )
    def mega(src_hbm, dst_hbm, g0_hbm, g1_hbm,
             dinv_hbm, gs0_hbm, gs1_hbm, o0_hbm, o1_hbm,
             d_v, s_v, deg, red, t0, t1, a0, a1,
             dv_s, g0s, g1s, sh_deg, sh_g0, sh_g1):
        sid = lax.axis_index("s")
        cid = lax.axis_index("c")

        _zero_tables([deg, a0, a1])
        dbase = pl.multiple_of(sid * EPC, 8)
        pltpu.sync_copy(dst_hbm.at[pl.ds(dbase, EPC)], d_v)
        ones = jnp.ones((L,), jnp.float32)

        @plsc.parallel_loop(0, EPC, step=L, unroll=8)
        def _(i):
            plsc.addupdate_scatter(deg, [d_v[pl.ds(i, L)]], ones)

        pltpu.sync_copy(deg, sh_deg.at[sid])
        plsc.subcore_barrier()

        off = pl.multiple_of(sid * SLICE, 8)
        for r in range(NSUB):
            pltpu.sync_copy(sh_deg.at[r, pl.ds(off, SLICE)],
                            red.at[pl.ds(r * SLICE, SLICE)])
        pltpu.sync_copy(g0_hbm.at[pl.ds(off, SLICE)], g0s)
        pltpu.sync_copy(g1_hbm.at[pl.ds(off, SLICE)], g1s)

        half = jnp.full((L,), 0.5, jnp.float32)
        three_half = jnp.full((L,), 1.5, jnp.float32)
        magic = jnp.full((L,), _RSQRT_MAGIC, jnp.int32)
        one_i = jnp.full((L,), 1, jnp.int32)

        @plsc.parallel_loop(0, SLICE, step=L, unroll=4)
        def _(j):
            s = red[pl.ds(j, L)]
            for r in range(1, NSUB):
                s = s + red[pl.ds(r * SLICE + j, L)]
            dd = s + 1.0
            y = plsc.bitcast(
                magic - lax.shift_right_logical(plsc.bitcast(dd, jnp.int32),
                                                one_i),
                jnp.float32)
            for _ in range(3):
                y = y * (three_half - half * dd * y * y)
            dv_s[pl.ds(j, L)] = y
            g0s[pl.ds(j, L)] = g0s[pl.ds(j, L)] * y
            g1s[pl.ds(j, L)] = g1s[pl.ds(j, L)] * y

        pltpu.sync_copy(g0s, sh_g0.at[pl.ds(off, SLICE)])
        pltpu.sync_copy(g1s, sh_g1.at[pl.ds(off, SLICE)])
        pltpu.sync_copy(dv_s, dinv_hbm.at[cid, pl.ds(off, SLICE)])
        pltpu.sync_copy(g0s, gs0_hbm.at[cid, pl.ds(off, SLICE)])
        pltpu.sync_copy(g1s, gs1_hbm.at[cid, pl.ds(off, SLICE)])
        plsc.subcore_barrier()
        pltpu.sync_copy(sh_g0, t0)
        pltpu.sync_copy(sh_g1, t1)

        sbase = pl.multiple_of(sid * EPC + cid * EPT, 8)
        pltpu.sync_copy(src_hbm.at[pl.ds(sbase, EPT)], s_v)
        soff = cid * EPT

        @plsc.parallel_loop(0, EPT, step=L, unroll=8)
        def _(i):
            s = s_v[pl.ds(i, L)]
            d = d_v[pl.ds(soff + i, L)]
            v0 = plsc.load_gather(t0, [s])
            plsc.addupdate_scatter(a0, [d], v0)
            v1 = plsc.load_gather(t1, [s])
            plsc.addupdate_scatter(a1, [d], v1)

        wid = cid * NSUB + sid
        pltpu.sync_copy(a0, o0_hbm.at[wid])
        pltpu.sync_copy(a1, o1_hbm.at[wid])

    return mega(src, dst, g0, g1)


def _tc_matmul(x, W1, W2, ei):
    def body(x_ref, w1_ref, w2_ref, e_ref, o0_ref, o1_ref, os_ref, od_ref):
        e = e_ref[...]
        os_ref[...] = e[0, :]
        od_ref[...] = e[1, :]
        wc = jnp.dot(w1_ref[...], w2_ref[...],
                     preferred_element_type=jnp.float32)
        gt = lax.dot_general(wc, x_ref[...], (((0,), (1,)), ((), ())),
                             preferred_element_type=jnp.float32)
        pad = jnp.zeros((1, NPAD - N_NODES), jnp.float32)
        o0_ref[...] = jnp.concatenate([gt[0:1, :], pad], axis=1)
        o1_ref[...] = jnp.concatenate([gt[1:2, :], pad], axis=1)

    return pl.pallas_call(
        body,
        out_shape=[
            jax.ShapeDtypeStruct((1, NPAD), jnp.float32),
            jax.ShapeDtypeStruct((1, NPAD), jnp.float32),
            jax.ShapeDtypeStruct((N_EDGES,), jnp.int32),
            jax.ShapeDtypeStruct((N_EDGES,), jnp.int32),
        ],
    )(x, W1, W2, ei)


def _tc_final(a0p, a1p, gs0, gs1, dinv2, b1r, W2, b2r):
    def body(a0_ref, a1_ref, gs0_ref, gs1_ref, dinv_ref, b1_ref, w2_ref,
             b2_ref, o_ref):
        bc = jnp.dot(b1_ref[...], w2_ref[...],
                     preferred_element_type=jnp.float32) + b2_ref[...]
        d = dinv_ref[0:1, :N_NODES]
        s0 = (jnp.sum(a0_ref[...], axis=0, keepdims=True)
              + gs0_ref[0:1, :])[:, :N_NODES]
        s1 = (jnp.sum(a1_ref[...], axis=0, keepdims=True)
              + gs1_ref[0:1, :])[:, :N_NODES]
        o_ref[...] = jnp.concatenate(
            [d * s0 + bc[:, 0:1], d * s1 + bc[:, 1:2]], axis=0)

    return pl.pallas_call(
        body,
        out_shape=jax.ShapeDtypeStruct((2, N_NODES), jnp.float32),
    )(a0p, a1p, gs0, gs1, dinv2, b1r, W2, b2r)


def kernel(x, edge_index, W1, b1, W2, b2):
    ei = edge_index.astype(jnp.int32)

    g0r, g1r, srcr, dstr = _tc_matmul(x, W1, W2, ei)
    dinv2, gs0, gs1, a0p, a1p = _sc_mega(
        srcr, dstr, g0r.reshape(NPAD), g1r.reshape(NPAD))

    out2 = _tc_final(a0p, a1p, gs0, gs1, dinv2,
                     b1.reshape(1, -1), W2, b2.reshape(1, -1))
    return out2.T
